# baseline (speedup 1.0000x reference)
import functools

import jax
import jax.numpy as jnp
import numpy as np
from jax import lax
from jax.experimental import pallas as pl
from jax.experimental.pallas import tpu as pltpu
from jax.experimental.pallas import tpu_sc as plsc

B, T, N = 2, 4, 4096
P = 1024
K = 32
C1, C2 = 32, 64
R2 = np.float32(0.3 * 0.3)
NCHUNK = N // 16
CAP = 64
BTN = B * T * N
BTP = B * T * P
NU = 10
PPT = 128
NUNIT = NU * B * (P // PPT)

_mesh = plsc.VectorSubcoreMesh(core_axis_name="c", subcore_axis_name="s")
_sc_params = pltpu.CompilerParams(needs_layout_passes=False)


QN = N // 4
QC = QN // 16


@functools.partial(
    pl.kernel,
    out_type=[
        jax.ShapeDtypeStruct((3 * BTN,), jnp.float32),
        jax.ShapeDtypeStruct((3 * BTP,), jnp.float32),
    ],
    mesh=_mesh,
    compiler_params=_sc_params,
    scratch_types=[
        pltpu.VMEM((3 * N,), jnp.float32),
        pltpu.VMEM((N,), jnp.float32),
        pltpu.VMEM((N,), jnp.float32),
        pltpu.VMEM((N,), jnp.float32),
        pltpu.VMEM((QN,), jnp.float32),
        pltpu.VMEM((P,), jnp.int32),
        pltpu.VMEM((P,), jnp.float32),
        pltpu.VMEM((P,), jnp.float32),
        pltpu.VMEM((P,), jnp.float32),
        pltpu.VMEM((32,), jnp.float32),
        pltpu.VMEM((512,), jnp.float32),
        pltpu.VMEM_SHARED((4096,), jnp.float32),
    ],
)
def _fps_kernel(xyzs, soa_out, axyz_out, aos_v, sx_v, sy_v, sz_v, dist_v,
                aidx_v, ax_v, ay_v, az_v, pkt_v, blk_v, shx_v):
    c_ax = lax.axis_index("c")
    s_ax = lax.axis_index("s")
    tl = s_ax // 4
    mem = s_ax % 4
    bt = c_ax * 4 + tl
    base = mem * QN
    iota = lax.iota(jnp.int32, 16)

    pltpu.sync_copy(xyzs.at[pl.ds(bt * 3 * N, 3 * N)], aos_v)

    def soa_body(c, _):
        rows = (c * 16 + iota) * 3
        for ch, ref in ((0, sx_v), (1, sy_v), (2, sz_v)):
            ref[pl.ds(c * 16, 16)] = plsc.load_gather(aos_v, [rows + ch])
        return 0

    lax.fori_loop(0, NCHUNK, soa_body, 0)

    def init_body(c, _):
        dist_v[pl.ds(c * 16, 16)] = jnp.full((16,), 1e10, jnp.float32)
        return 0

    lax.fori_loop(0, QC, init_body, 0)
    lane0 = iota == 0
    gchunk0 = mem * QC

    def step(i, far):
        farv = jnp.full((16,), far, jnp.int32)
        plsc.store_scatter(aidx_v, [jnp.full((16,), i, jnp.int32)],
                           farv, mask=lane0)
        cx = plsc.load_gather(sx_v, [farv])
        cy = plsc.load_gather(sy_v, [farv])
        cz = plsc.load_gather(sz_v, [farv])

        def chunk(c, carry):
            m, mi = carry
            s = pl.ds(base + c * 16, 16)
            dx = sx_v[s] - cx
            dy = sy_v[s] - cy
            dz = sz_v[s] - cz
            d2 = (jnp.minimum(dx * dx, 3.0) + jnp.minimum(dz * dz, 3.0)
                  ) + jnp.minimum(dy * dy, 3.0)
            dv = jnp.minimum(dist_v[pl.ds(c * 16, 16)], d2)
            dist_v[pl.ds(c * 16, 16)] = dv
            upd = dv > m
            m = jnp.where(upd, dv, m)
            mi = jnp.where(upd, jnp.full((16,), c, jnp.int32), mi)
            return m, mi

        m0 = jnp.full((16,), -1.0, jnp.float32)
        i0 = jnp.zeros((16,), jnp.int32)
        m, mi = lax.fori_loop(0, QC, chunk, (m0, i0), unroll=4)
        gidx = (mi + gchunk0) * 16 + iota
        pkt_v[pl.ds(0, 16)] = m
        pkt_v[pl.ds(16, 16)] = plsc.bitcast(gidx, jnp.float32)
        parity = (i % 2) * 2048
        pltpu.sync_copy(pkt_v, shx_v.at[pl.ds(parity + s_ax * 128, 32)])
        plsc.subcore_barrier()
        pltpu.sync_copy(shx_v.at[pl.ds(parity + tl * 512, 512)], blk_v)
        vals = [blk_v[pl.ds(j * 128, 16)] for j in range(4)]
        idxs = [plsc.bitcast(blk_v[pl.ds(j * 128 + 16, 16)], jnp.int32)
                for j in range(4)]
        mm = jnp.maximum(jnp.maximum(vals[0], vals[1]),
                         jnp.maximum(vals[2], vals[3]))
        mx = jnp.max(mm)
        big = jnp.int32(1 << 30)
        cands = [jnp.where(vals[j] == mx, idxs[j], big) for j in range(4)]
        cand = jnp.minimum(jnp.minimum(cands[0], cands[1]),
                           jnp.minimum(cands[2], cands[3]))
        return jnp.min(cand)

    lax.fori_loop(0, P, step, jnp.int32(0))

    @pl.when(mem == 0)
    def _():
        def gath(c, _):
            s = pl.ds(c * 16, 16)
            idxs = aidx_v[s]
            ax_v[s] = plsc.load_gather(sx_v, [idxs])
            ay_v[s] = plsc.load_gather(sy_v, [idxs])
            az_v[s] = plsc.load_gather(sz_v, [idxs])
            return 0

        lax.fori_loop(0, P // 16, gath, 0)

        for ch, (src, asrc) in enumerate(((sx_v, ax_v), (sy_v, ay_v),
                                          (sz_v, az_v))):
            pltpu.sync_copy(src, soa_out.at[pl.ds(ch * BTN + bt * N, N)])
            pltpu.sync_copy(asrc, axyz_out.at[pl.ds(ch * BTP + bt * P, P)])


@functools.partial(
    pl.kernel,
    out_type=jax.ShapeDtypeStruct((NU * B * 3 * K, P), jnp.float32),
    mesh=_mesh,
    compiler_params=_sc_params,
    scratch_types=[
        pltpu.VMEM((N,), jnp.float32),
        pltpu.VMEM((N,), jnp.float32),
        pltpu.VMEM((N,), jnp.float32),
        pltpu.VMEM((PPT,), jnp.float32),
        pltpu.VMEM((PPT,), jnp.float32),
        pltpu.VMEM((PPT,), jnp.float32),
        pltpu.VMEM((CAP,), jnp.float32),
        pltpu.VMEM((CAP,), jnp.float32),
        pltpu.VMEM((CAP,), jnp.float32),
        pltpu.VMEM((3 * K, PPT), jnp.float32),
    ],
)
def _bq_kernel(soa, axyz, disp_out, nx_v, ny_v, nz_v,
               anx_v, any_v, anz_v, cpx_v, cpy_v, cpz_v, stg_v):
    wid = lax.axis_index("s") * 2 + lax.axis_index("c")
    iota = lax.iota(jnp.int32, 16)

    def job(j, _):
        unit = j * 32 + wid
        q = unit // (P // PPT)
        p0 = (unit % (P // PPT)) * PPT
        b = q % 2
        qu = q // 2
        a = (qu + 1) // 3
        u = qu - 2 * a
        pltpu.sync_copy(soa.at[pl.ds((b * T + u) * N, N)], nx_v)
        pltpu.sync_copy(soa.at[pl.ds(BTN + (b * T + u) * N, N)], ny_v)
        pltpu.sync_copy(soa.at[pl.ds(2 * BTN + (b * T + u) * N, N)], nz_v)
        abase = (b * T + a) * P + p0
        pltpu.sync_copy(axyz.at[pl.ds(abase, PPT)], anx_v)
        pltpu.sync_copy(axyz.at[pl.ds(BTP + abase, PPT)], any_v)
        pltpu.sync_copy(axyz.at[pl.ds(2 * BTP + abase, PPT)], anz_v)

        def anchor(p, _):
            pv = jnp.full((16,), p, jnp.int32)
            ax = plsc.load_gather(anx_v, [pv])
            ay = plsc.load_gather(any_v, [pv])
            az = plsc.load_gather(anz_v, [pv])

            def cond(carry):
                cnt, c = carry
                return (cnt < K) & (c < NCHUNK)

            def scan(carry):
                cnt, c = carry
                for half in range(2):
                    s = pl.ds((c + half) * 16, 16)
                    dx = nx_v[s] - ax
                    dy = ny_v[s] - ay
                    dz = nz_v[s] - az
                    d2 = (jnp.minimum(dx * dx, 3.0)
                          + jnp.minimum(dy * dy, 3.0)
                          ) + jnp.minimum(dz * dz, 3.0)
                    msk = d2 < R2
                    plsc.store_compressed(cpx_v.at[pl.ds(cnt, 16)], dx,
                                          mask=msk)
                    plsc.store_compressed(cpy_v.at[pl.ds(cnt, 16)], dy,
                                          mask=msk)
                    plsc.store_compressed(cpz_v.at[pl.ds(cnt, 16)], dz,
                                          mask=msk)
                    cnt = cnt + plsc.all_reduce_population_count(msk)[0]
                return cnt, c + 2

            cnt, _c = lax.while_loop(cond, scan, (jnp.int32(0), jnp.int32(0)))

            @pl.when(cnt < K)
            def _():
                zero16 = jnp.zeros((16,), jnp.int32)
                hasv = jnp.broadcast_to(cnt > 0, (16,))
                cntv = jnp.broadcast_to(cnt, (16,))
                pdx = jnp.where(hasv, plsc.load_gather(cpx_v, [zero16]),
                                plsc.load_gather(nx_v, [zero16]) - ax)
                pdy = jnp.where(hasv, plsc.load_gather(cpy_v, [zero16]),
                                plsc.load_gather(ny_v, [zero16]) - ay)
                pdz = jnp.where(hasv, plsc.load_gather(cpz_v, [zero16]),
                                plsc.load_gather(nz_v, [zero16]) - az)
                for chunk in range(K // 16):
                    lanes = iota + chunk * 16
                    fill = lanes >= cntv
                    for ref, padv in ((cpx_v, pdx), (cpy_v, pdy),
                                      (cpz_v, pdz)):
                        v = ref[pl.ds(chunk * 16, 16)]
                        ref[pl.ds(chunk * 16, 16)] = jnp.where(fill, padv, v)

            for chunk in range(K // 16):
                lanes = iota + chunk * 16
                for ch, ref in enumerate((cpx_v, cpy_v, cpz_v)):
                    rows = lanes + ch * K
                    plsc.store_scatter(stg_v, [rows, pv],
                                       ref[pl.ds(chunk * 16, 16)])
            return 0

        lax.fori_loop(0, PPT, anchor, 0)
        row0 = q * 3 * K
        pltpu.sync_copy(stg_v,
                        disp_out.at[pl.ds(row0, 3 * K), pl.ds(p0, PPT)])
        return 0

    lax.fori_loop(0, NUNIT // 32, job, 0)


def _uq_lookup(t, w):
    return jnp.clip(t * 3 + w - 1, 0, NU - 1)


def _mlp_body(d_ref, wd_ref, wm_ref, o_ref):
    w = pl.program_id(2)
    x = d_ref[0, 0]
    wd = wd_ref[...]
    bias = (w.astype(jnp.float32) - 1.0) * wd[:, 3:4]
    h = (wd[:, 0:1] * x[0:1, :] + wd[:, 1:2] * x[1:2, :]
         + wd[:, 2:3] * x[2:3, :] + bias)
    h = jnp.maximum(h, 0.0)
    f = jnp.dot(wm_ref[...].astype(jnp.bfloat16), h.astype(jnp.bfloat16),
                preferred_element_type=jnp.float32)
    f = jnp.maximum(f, 0.0)
    mx = f[:, 0:P]
    for k in range(1, K):
        mx = jnp.maximum(mx, f[:, k * P:(k + 1) * P])

    @pl.when(w == 0)
    def _():
        o_ref[0, 0] = mx

    @pl.when(w > 0)
    def _():
        o_ref[0, 0] += mx


def _mlp_call(dispf, Wd, Wm):
    return pl.pallas_call(
        _mlp_body,
        grid=(B, T, 3),
        in_specs=[
            pl.BlockSpec((1, 1, 3, K * P),
                         lambda b, t, w: (_uq_lookup(t, w), b, 0, 0)),
            pl.BlockSpec((C1, 4), lambda b, t, w: (0, 0)),
            pl.BlockSpec((C2, C1), lambda b, t, w: (0, 0)),
        ],
        out_specs=pl.BlockSpec((1, 1, C2, P), lambda b, t, w: (b, t, 0, 0)),
        out_shape=jax.ShapeDtypeStruct((B, T, C2, P), jnp.float32),
    )(dispf, Wd, Wm)


def kernel(xyzs, Wd, Wm):
    soa, axyz = _fps_kernel(xyzs.reshape(-1))
    disp = _bq_kernel(soa, axyz)
    feats = _mlp_call(disp.reshape(NU, B, 3, K * P), Wd, Wm)
    new_xyzs = jnp.transpose(axyz.reshape(3, B, T, P), (1, 2, 3, 0))
    return new_xyzs, feats


# speedup vs baseline: 1.0012x; 1.0012x over previous
import functools

import jax
import jax.numpy as jnp
import numpy as np
from jax import lax
from jax.experimental import pallas as pl
from jax.experimental.pallas import tpu as pltpu
from jax.experimental.pallas import tpu_sc as plsc

B, T, N = 2, 4, 4096
P = 1024
K = 32
C1, C2 = 32, 64
R2 = np.float32(0.3 * 0.3)
NCHUNK = N // 16
CAP = 64
BTN = B * T * N
BTP = B * T * P
NU = 10
PPT = 128
NUNIT = NU * B * (P // PPT)

_mesh = plsc.VectorSubcoreMesh(core_axis_name="c", subcore_axis_name="s")
_sc_params = pltpu.CompilerParams(needs_layout_passes=False)


QN = N // 4
QC = QN // 16


@functools.partial(
    pl.kernel,
    out_type=[
        jax.ShapeDtypeStruct((3 * BTN,), jnp.float32),
        jax.ShapeDtypeStruct((3 * BTP,), jnp.float32),
    ],
    mesh=_mesh,
    compiler_params=_sc_params,
    scratch_types=[
        pltpu.VMEM((3 * N,), jnp.float32),
        pltpu.VMEM((N,), jnp.float32),
        pltpu.VMEM((N,), jnp.float32),
        pltpu.VMEM((N,), jnp.float32),
        pltpu.VMEM((QN,), jnp.float32),
        pltpu.VMEM((P,), jnp.int32),
        pltpu.VMEM((P,), jnp.float32),
        pltpu.VMEM((P,), jnp.float32),
        pltpu.VMEM((P,), jnp.float32),
        pltpu.VMEM((32,), jnp.float32),
        pltpu.VMEM((512,), jnp.float32),
        pltpu.VMEM_SHARED((4096,), jnp.float32),
    ],
)
def _fps_kernel(xyzs, soa_out, axyz_out, aos_v, sx_v, sy_v, sz_v, dist_v,
                aidx_v, ax_v, ay_v, az_v, pkt_v, blk_v, shx_v):
    c_ax = lax.axis_index("c")
    s_ax = lax.axis_index("s")
    tl = s_ax // 4
    mem = s_ax % 4
    bt = c_ax * 4 + tl
    base = mem * QN
    iota = lax.iota(jnp.int32, 16)

    pltpu.sync_copy(xyzs.at[pl.ds(bt * 3 * N, 3 * N)], aos_v)

    def soa_body(c, _):
        rows = (c * 16 + iota) * 3
        for ch, ref in ((0, sx_v), (1, sy_v), (2, sz_v)):
            ref[pl.ds(c * 16, 16)] = plsc.load_gather(aos_v, [rows + ch])
        return 0

    lax.fori_loop(0, NCHUNK, soa_body, 0)

    def init_body(c, _):
        dist_v[pl.ds(c * 16, 16)] = jnp.full((16,), 1e10, jnp.float32)
        return 0

    lax.fori_loop(0, QC, init_body, 0)
    lane0 = iota == 0
    gchunk0 = mem * QC

    def step(i, far):
        farv = jnp.full((16,), far, jnp.int32)
        plsc.store_scatter(aidx_v, [jnp.full((16,), i, jnp.int32)],
                           farv, mask=lane0)
        cx = plsc.load_gather(sx_v, [farv])
        cy = plsc.load_gather(sy_v, [farv])
        cz = plsc.load_gather(sz_v, [farv])

        def chunk(c, carry):
            m, mi = carry
            s = pl.ds(base + c * 16, 16)
            dx = sx_v[s] - cx
            dy = sy_v[s] - cy
            dz = sz_v[s] - cz
            d2 = (jnp.minimum(dx * dx, 3.0) + jnp.minimum(dz * dz, 3.0)
                  ) + jnp.minimum(dy * dy, 3.0)
            dv = jnp.minimum(dist_v[pl.ds(c * 16, 16)], d2)
            dist_v[pl.ds(c * 16, 16)] = dv
            upd = dv > m
            m = jnp.where(upd, dv, m)
            mi = jnp.where(upd, jnp.full((16,), c, jnp.int32), mi)
            return m, mi

        m0 = jnp.full((16,), -1.0, jnp.float32)
        i0 = jnp.zeros((16,), jnp.int32)
        m, mi = lax.fori_loop(0, QC, chunk, (m0, i0), unroll=4)
        gidx = (mi + gchunk0) * 16 + iota
        pkt_v[pl.ds(0, 16)] = m
        pkt_v[pl.ds(16, 16)] = plsc.bitcast(gidx, jnp.float32)
        parity = (i % 2) * 2048
        pltpu.sync_copy(pkt_v, shx_v.at[pl.ds(parity + s_ax * 128, 32)])
        plsc.subcore_barrier()
        pltpu.sync_copy(shx_v.at[pl.ds(parity + tl * 512, 512)], blk_v)
        vals = [blk_v[pl.ds(j * 128, 16)] for j in range(4)]
        idxs = [plsc.bitcast(blk_v[pl.ds(j * 128 + 16, 16)], jnp.int32)
                for j in range(4)]
        mm = jnp.maximum(jnp.maximum(vals[0], vals[1]),
                         jnp.maximum(vals[2], vals[3]))
        mx = jnp.max(mm)
        big = jnp.int32(1 << 30)
        cands = [jnp.where(vals[j] == mx, idxs[j], big) for j in range(4)]
        cand = jnp.minimum(jnp.minimum(cands[0], cands[1]),
                           jnp.minimum(cands[2], cands[3]))
        return jnp.min(cand)

    lax.fori_loop(0, P, step, jnp.int32(0))

    @pl.when(mem == 0)
    def _():
        def gath(c, _):
            s = pl.ds(c * 16, 16)
            idxs = aidx_v[s]
            ax_v[s] = plsc.load_gather(sx_v, [idxs])
            ay_v[s] = plsc.load_gather(sy_v, [idxs])
            az_v[s] = plsc.load_gather(sz_v, [idxs])
            return 0

        lax.fori_loop(0, P // 16, gath, 0)

        for ch, (src, asrc) in enumerate(((sx_v, ax_v), (sy_v, ay_v),
                                          (sz_v, az_v))):
            pltpu.sync_copy(src, soa_out.at[pl.ds(ch * BTN + bt * N, N)])
            pltpu.sync_copy(asrc, axyz_out.at[pl.ds(ch * BTP + bt * P, P)])


@functools.partial(
    pl.kernel,
    out_type=jax.ShapeDtypeStruct((NU * B * 3 * K, P), jnp.float32),
    mesh=_mesh,
    compiler_params=_sc_params,
    scratch_types=[
        pltpu.VMEM((N,), jnp.float32),
        pltpu.VMEM((N,), jnp.float32),
        pltpu.VMEM((N,), jnp.float32),
        pltpu.VMEM((PPT,), jnp.float32),
        pltpu.VMEM((PPT,), jnp.float32),
        pltpu.VMEM((PPT,), jnp.float32),
        pltpu.VMEM((CAP,), jnp.float32),
        pltpu.VMEM((CAP,), jnp.float32),
        pltpu.VMEM((CAP,), jnp.float32),
        pltpu.VMEM((3 * K, PPT), jnp.float32),
    ],
)
def _bq_kernel(soa, axyz, disp_out, nx_v, ny_v, nz_v,
               anx_v, any_v, anz_v, cpx_v, cpy_v, cpz_v, stg_v):
    wid = lax.axis_index("s") * 2 + lax.axis_index("c")
    iota = lax.iota(jnp.int32, 16)

    def job(j, _):
        unit = j * 32 + wid
        q = unit // (P // PPT)
        p0 = (unit % (P // PPT)) * PPT
        b = q % 2
        qu = q // 2
        a = (qu + 1) // 3
        u = qu - 2 * a
        pltpu.sync_copy(soa.at[pl.ds((b * T + u) * N, N)], nx_v)
        pltpu.sync_copy(soa.at[pl.ds(BTN + (b * T + u) * N, N)], ny_v)
        pltpu.sync_copy(soa.at[pl.ds(2 * BTN + (b * T + u) * N, N)], nz_v)
        abase = (b * T + a) * P + p0
        pltpu.sync_copy(axyz.at[pl.ds(abase, PPT)], anx_v)
        pltpu.sync_copy(axyz.at[pl.ds(BTP + abase, PPT)], any_v)
        pltpu.sync_copy(axyz.at[pl.ds(2 * BTP + abase, PPT)], anz_v)

        def anchor(p, _):
            pv = jnp.full((16,), p, jnp.int32)
            ax = plsc.load_gather(anx_v, [pv])
            ay = plsc.load_gather(any_v, [pv])
            az = plsc.load_gather(anz_v, [pv])

            def cond(carry):
                cnt, c = carry
                return (cnt < K) & (c < NCHUNK)

            def scan(carry):
                cnt, c = carry
                for half in range(2):
                    s = pl.ds((c + half) * 16, 16)
                    dx = nx_v[s] - ax
                    dy = ny_v[s] - ay
                    dz = nz_v[s] - az
                    d2 = (jnp.minimum(dx * dx, 3.0)
                          + jnp.minimum(dy * dy, 3.0)
                          ) + jnp.minimum(dz * dz, 3.0)
                    msk = d2 < R2
                    plsc.store_compressed(cpx_v.at[pl.ds(cnt, 16)], dx,
                                          mask=msk)
                    plsc.store_compressed(cpy_v.at[pl.ds(cnt, 16)], dy,
                                          mask=msk)
                    plsc.store_compressed(cpz_v.at[pl.ds(cnt, 16)], dz,
                                          mask=msk)
                    cnt = cnt + plsc.all_reduce_population_count(msk)[0]
                return cnt, c + 2

            cnt, _c = lax.while_loop(cond, scan, (jnp.int32(0), jnp.int32(0)))
            zero16 = jnp.zeros((16,), jnp.int32)
            hasv = jnp.broadcast_to(cnt > 0, (16,))
            cntv = jnp.broadcast_to(cnt, (16,))
            pdx = jnp.where(hasv, plsc.load_gather(cpx_v, [zero16]),
                            plsc.load_gather(nx_v, [zero16]) - ax)
            pdy = jnp.where(hasv, plsc.load_gather(cpy_v, [zero16]),
                            plsc.load_gather(ny_v, [zero16]) - ay)
            pdz = jnp.where(hasv, plsc.load_gather(cpz_v, [zero16]),
                            plsc.load_gather(nz_v, [zero16]) - az)
            for chunk in range(K // 16):
                lanes = iota + chunk * 16
                fill = lanes >= cntv
                for ch, (ref, padv) in enumerate(((cpx_v, pdx), (cpy_v, pdy),
                                                  (cpz_v, pdz))):
                    v = ref[pl.ds(chunk * 16, 16)]
                    v = jnp.where(fill, padv, v)
                    rows = lanes + ch * K
                    plsc.store_scatter(stg_v, [rows, pv], v)
            return 0

        lax.fori_loop(0, PPT, anchor, 0)
        row0 = q * 3 * K
        pltpu.sync_copy(stg_v,
                        disp_out.at[pl.ds(row0, 3 * K), pl.ds(p0, PPT)])
        return 0

    lax.fori_loop(0, NUNIT // 32, job, 0)


def _uq_lookup(t, w):
    return jnp.clip(t * 3 + w - 1, 0, NU - 1)


def _mlp_body(d_ref, wd_ref, wm_ref, o_ref):
    w = pl.program_id(2)
    x = d_ref[0, 0]
    wd = wd_ref[...]
    bias = (w.astype(jnp.float32) - 1.0) * wd[:, 3:4]
    h = (wd[:, 0:1] * x[0:1, :] + wd[:, 1:2] * x[1:2, :]
         + wd[:, 2:3] * x[2:3, :] + bias)
    h = jnp.maximum(h, 0.0)
    f = jnp.dot(wm_ref[...].astype(jnp.bfloat16), h.astype(jnp.bfloat16),
                preferred_element_type=jnp.float32)
    f = jnp.maximum(f, 0.0)
    mx = f[:, 0:P]
    for k in range(1, K):
        mx = jnp.maximum(mx, f[:, k * P:(k + 1) * P])

    @pl.when(w == 0)
    def _():
        o_ref[0, 0] = mx

    @pl.when(w > 0)
    def _():
        o_ref[0, 0] += mx


def _mlp_call(dispf, Wd, Wm):
    return pl.pallas_call(
        _mlp_body,
        grid=(B, T, 3),
        in_specs=[
            pl.BlockSpec((1, 1, 3, K * P),
                         lambda b, t, w: (_uq_lookup(t, w), b, 0, 0)),
            pl.BlockSpec((C1, 4), lambda b, t, w: (0, 0)),
            pl.BlockSpec((C2, C1), lambda b, t, w: (0, 0)),
        ],
        out_specs=pl.BlockSpec((1, 1, C2, P), lambda b, t, w: (b, t, 0, 0)),
        out_shape=jax.ShapeDtypeStruct((B, T, C2, P), jnp.float32),
    )(dispf, Wd, Wm)


def kernel(xyzs, Wd, Wm):
    soa, axyz = _fps_kernel(xyzs.reshape(-1))
    disp = _bq_kernel(soa, axyz)
    feats = _mlp_call(disp.reshape(NU, B, 3, K * P), Wd, Wm)
    new_xyzs = jnp.transpose(axyz.reshape(3, B, T, P), (1, 2, 3, 0))
    return new_xyzs, feats


# speedup vs baseline: 1.0155x; 1.0142x over previous
import functools

import jax
import jax.numpy as jnp
import numpy as np
from jax import lax
from jax.experimental import pallas as pl
from jax.experimental.pallas import tpu as pltpu
from jax.experimental.pallas import tpu_sc as plsc

B, T, N = 2, 4, 4096
P = 1024
K = 32
C1, C2 = 32, 64
R2 = np.float32(0.3 * 0.3)
NCHUNK = N // 16
CAP = 64
BTN = B * T * N
BTP = B * T * P
NU = 10
PPT = 128
NUNIT = NU * B * (P // PPT)

_mesh = plsc.VectorSubcoreMesh(core_axis_name="c", subcore_axis_name="s")
_sc_params = pltpu.CompilerParams(needs_layout_passes=False)


QN = N // 4
QC = QN // 16


@functools.partial(
    pl.kernel,
    out_type=[
        jax.ShapeDtypeStruct((3 * BTN,), jnp.float32),
        jax.ShapeDtypeStruct((3 * BTP,), jnp.float32),
    ],
    mesh=_mesh,
    compiler_params=_sc_params,
    scratch_types=[
        pltpu.VMEM((3 * N,), jnp.float32),
        pltpu.VMEM((N,), jnp.float32),
        pltpu.VMEM((N,), jnp.float32),
        pltpu.VMEM((N,), jnp.float32),
        pltpu.VMEM((QN,), jnp.float32),
        pltpu.VMEM((P,), jnp.int32),
        pltpu.VMEM((P,), jnp.float32),
        pltpu.VMEM((P,), jnp.float32),
        pltpu.VMEM((P,), jnp.float32),
        pltpu.VMEM((32,), jnp.float32),
        pltpu.VMEM((512,), jnp.float32),
        pltpu.VMEM_SHARED((4096,), jnp.float32),
    ],
)
def _fps_kernel(xyzs, soa_out, axyz_out, aos_v, sx_v, sy_v, sz_v, dist_v,
                aidx_v, ax_v, ay_v, az_v, pkt_v, blk_v, shx_v):
    c_ax = lax.axis_index("c")
    s_ax = lax.axis_index("s")
    tl = s_ax // 4
    mem = s_ax % 4
    bt = c_ax * 4 + tl
    base = mem * QN
    iota = lax.iota(jnp.int32, 16)

    pltpu.sync_copy(xyzs.at[pl.ds(bt * 3 * N, 3 * N)], aos_v)

    def soa_body(c, _):
        rows = (c * 16 + iota) * 3
        for ch, ref in ((0, sx_v), (1, sy_v), (2, sz_v)):
            ref[pl.ds(c * 16, 16)] = plsc.load_gather(aos_v, [rows + ch])
        return 0

    lax.fori_loop(0, NCHUNK, soa_body, 0)

    def init_body(c, _):
        dist_v[pl.ds(c * 16, 16)] = jnp.full((16,), 1e10, jnp.float32)
        return 0

    lax.fori_loop(0, QC, init_body, 0)
    lane0 = iota == 0
    gchunk0 = mem * QC

    def step(i, far):
        farv = jnp.full((16,), far, jnp.int32)
        plsc.store_scatter(aidx_v, [jnp.full((16,), i, jnp.int32)],
                           farv, mask=lane0)
        cx = plsc.load_gather(sx_v, [farv])
        cy = plsc.load_gather(sy_v, [farv])
        cz = plsc.load_gather(sz_v, [farv])

        def chunk(c, carry):
            m, mi = carry
            s = pl.ds(base + c * 16, 16)
            dx = sx_v[s] - cx
            dy = sy_v[s] - cy
            dz = sz_v[s] - cz
            d2 = (jnp.minimum(dx * dx, 3.0) + jnp.minimum(dz * dz, 3.0)
                  ) + jnp.minimum(dy * dy, 3.0)
            dv = jnp.minimum(dist_v[pl.ds(c * 16, 16)], d2)
            dist_v[pl.ds(c * 16, 16)] = dv
            upd = dv > m
            m = jnp.where(upd, dv, m)
            mi = jnp.where(upd, jnp.full((16,), c, jnp.int32), mi)
            return m, mi

        m0 = jnp.full((16,), -1.0, jnp.float32)
        i0 = jnp.zeros((16,), jnp.int32)
        m, mi = lax.fori_loop(0, QC, chunk, (m0, i0), unroll=4)
        gidx = (mi + gchunk0) * 16 + iota
        pkt_v[pl.ds(0, 16)] = m
        pkt_v[pl.ds(16, 16)] = plsc.bitcast(gidx, jnp.float32)
        parity = (i % 2) * 2048
        pltpu.sync_copy(pkt_v,
                        shx_v.at[pl.ds(parity + tl * 512 + mem * 32, 32)])
        plsc.subcore_barrier()
        pltpu.sync_copy(shx_v.at[pl.ds(parity + tl * 512, 128)],
                        blk_v.at[pl.ds(0, 128)])
        vals = [blk_v[pl.ds(j * 32, 16)] for j in range(4)]
        idxs = [plsc.bitcast(blk_v[pl.ds(j * 32 + 16, 16)], jnp.int32)
                for j in range(4)]
        mm = jnp.maximum(jnp.maximum(vals[0], vals[1]),
                         jnp.maximum(vals[2], vals[3]))
        mx = jnp.max(mm)
        big = jnp.int32(1 << 30)
        cands = [jnp.where(vals[j] == mx, idxs[j], big) for j in range(4)]
        cand = jnp.minimum(jnp.minimum(cands[0], cands[1]),
                           jnp.minimum(cands[2], cands[3]))
        return jnp.min(cand)

    lax.fori_loop(0, P, step, jnp.int32(0))

    @pl.when(mem == 0)
    def _():
        def gath(c, _):
            s = pl.ds(c * 16, 16)
            idxs = aidx_v[s]
            ax_v[s] = plsc.load_gather(sx_v, [idxs])
            ay_v[s] = plsc.load_gather(sy_v, [idxs])
            az_v[s] = plsc.load_gather(sz_v, [idxs])
            return 0

        lax.fori_loop(0, P // 16, gath, 0)

        for ch, (src, asrc) in enumerate(((sx_v, ax_v), (sy_v, ay_v),
                                          (sz_v, az_v))):
            pltpu.sync_copy(src, soa_out.at[pl.ds(ch * BTN + bt * N, N)])
            pltpu.sync_copy(asrc, axyz_out.at[pl.ds(ch * BTP + bt * P, P)])


@functools.partial(
    pl.kernel,
    out_type=jax.ShapeDtypeStruct((NU * B * 3 * K, P), jnp.float32),
    mesh=_mesh,
    compiler_params=_sc_params,
    scratch_types=[
        pltpu.VMEM((N,), jnp.float32),
        pltpu.VMEM((N,), jnp.float32),
        pltpu.VMEM((N,), jnp.float32),
        pltpu.VMEM((PPT,), jnp.float32),
        pltpu.VMEM((PPT,), jnp.float32),
        pltpu.VMEM((PPT,), jnp.float32),
        pltpu.VMEM((CAP,), jnp.float32),
        pltpu.VMEM((CAP,), jnp.float32),
        pltpu.VMEM((CAP,), jnp.float32),
        pltpu.VMEM((3 * K, PPT), jnp.float32),
    ],
)
def _bq_kernel(soa, axyz, disp_out, nx_v, ny_v, nz_v,
               anx_v, any_v, anz_v, cpx_v, cpy_v, cpz_v, stg_v):
    wid = lax.axis_index("s") * 2 + lax.axis_index("c")
    iota = lax.iota(jnp.int32, 16)

    def job(j, _):
        unit = j * 32 + wid
        q = unit // (P // PPT)
        p0 = (unit % (P // PPT)) * PPT
        b = q % 2
        qu = q // 2
        a = (qu + 1) // 3
        u = qu - 2 * a
        pltpu.sync_copy(soa.at[pl.ds((b * T + u) * N, N)], nx_v)
        pltpu.sync_copy(soa.at[pl.ds(BTN + (b * T + u) * N, N)], ny_v)
        pltpu.sync_copy(soa.at[pl.ds(2 * BTN + (b * T + u) * N, N)], nz_v)
        abase = (b * T + a) * P + p0
        pltpu.sync_copy(axyz.at[pl.ds(abase, PPT)], anx_v)
        pltpu.sync_copy(axyz.at[pl.ds(BTP + abase, PPT)], any_v)
        pltpu.sync_copy(axyz.at[pl.ds(2 * BTP + abase, PPT)], anz_v)

        def anchor(p, _):
            pv = jnp.full((16,), p, jnp.int32)
            ax = plsc.load_gather(anx_v, [pv])
            ay = plsc.load_gather(any_v, [pv])
            az = plsc.load_gather(anz_v, [pv])

            def cond(carry):
                cnt, c = carry
                return (cnt < K) & (c < NCHUNK)

            def scan(carry):
                cnt, c = carry
                for half in range(2):
                    s = pl.ds((c + half) * 16, 16)
                    dx = nx_v[s] - ax
                    dy = ny_v[s] - ay
                    dz = nz_v[s] - az
                    d2 = (jnp.minimum(dx * dx, 3.0)
                          + jnp.minimum(dy * dy, 3.0)
                          ) + jnp.minimum(dz * dz, 3.0)
                    msk = d2 < R2
                    plsc.store_compressed(cpx_v.at[pl.ds(cnt, 16)], dx,
                                          mask=msk)
                    plsc.store_compressed(cpy_v.at[pl.ds(cnt, 16)], dy,
                                          mask=msk)
                    plsc.store_compressed(cpz_v.at[pl.ds(cnt, 16)], dz,
                                          mask=msk)
                    cnt = cnt + plsc.all_reduce_population_count(msk)[0]
                return cnt, c + 2

            cnt, _c = lax.while_loop(cond, scan, (jnp.int32(0), jnp.int32(0)))
            zero16 = jnp.zeros((16,), jnp.int32)
            hasv = jnp.broadcast_to(cnt > 0, (16,))
            cntv = jnp.broadcast_to(cnt, (16,))
            pdx = jnp.where(hasv, plsc.load_gather(cpx_v, [zero16]),
                            plsc.load_gather(nx_v, [zero16]) - ax)
            pdy = jnp.where(hasv, plsc.load_gather(cpy_v, [zero16]),
                            plsc.load_gather(ny_v, [zero16]) - ay)
            pdz = jnp.where(hasv, plsc.load_gather(cpz_v, [zero16]),
                            plsc.load_gather(nz_v, [zero16]) - az)
            for chunk in range(K // 16):
                lanes = iota + chunk * 16
                fill = lanes >= cntv
                for ch, (ref, padv) in enumerate(((cpx_v, pdx), (cpy_v, pdy),
                                                  (cpz_v, pdz))):
                    v = ref[pl.ds(chunk * 16, 16)]
                    v = jnp.where(fill, padv, v)
                    rows = lanes + ch * K
                    plsc.store_scatter(stg_v, [rows, pv], v)
            return 0

        lax.fori_loop(0, PPT, anchor, 0)
        row0 = q * 3 * K
        pltpu.sync_copy(stg_v,
                        disp_out.at[pl.ds(row0, 3 * K), pl.ds(p0, PPT)])
        return 0

    lax.fori_loop(0, NUNIT // 32, job, 0)


def _uq_lookup(t, w):
    return jnp.clip(t * 3 + w - 1, 0, NU - 1)


def _mlp_body(d_ref, wd_ref, wm_ref, o_ref):
    w = pl.program_id(2)
    x = d_ref[0, 0]
    wd = wd_ref[...]
    bias = (w.astype(jnp.float32) - 1.0) * wd[:, 3:4]
    h = (wd[:, 0:1] * x[0:1, :] + wd[:, 1:2] * x[1:2, :]
         + wd[:, 2:3] * x[2:3, :] + bias)
    h = jnp.maximum(h, 0.0)
    f = jnp.dot(wm_ref[...], h, preferred_element_type=jnp.float32)
    f = jnp.maximum(f, 0.0)
    mx = f[:, 0:P]
    for k in range(1, K):
        mx = jnp.maximum(mx, f[:, k * P:(k + 1) * P])

    @pl.when(w == 0)
    def _():
        o_ref[0, 0] = mx

    @pl.when(w > 0)
    def _():
        o_ref[0, 0] += mx


def _mlp_call(dispf, Wd, Wm):
    return pl.pallas_call(
        _mlp_body,
        grid=(B, T, 3),
        in_specs=[
            pl.BlockSpec((1, 1, 3, K * P),
                         lambda b, t, w: (_uq_lookup(t, w), b, 0, 0)),
            pl.BlockSpec((C1, 4), lambda b, t, w: (0, 0)),
            pl.BlockSpec((C2, C1), lambda b, t, w: (0, 0)),
        ],
        out_specs=pl.BlockSpec((1, 1, C2, P), lambda b, t, w: (b, t, 0, 0)),
        out_shape=jax.ShapeDtypeStruct((B, T, C2, P), jnp.float32),
    )(dispf, Wd, Wm)


def kernel(xyzs, Wd, Wm):
    soa, axyz = _fps_kernel(xyzs.reshape(-1))
    disp = _bq_kernel(soa, axyz)
    feats = _mlp_call(disp.reshape(NU, B, 3, K * P), Wd, Wm)
    new_xyzs = jnp.transpose(axyz.reshape(3, B, T, P), (1, 2, 3, 0))
    return new_xyzs, feats


# speedup vs baseline: 1.0457x; 1.0298x over previous
import functools

import jax
import jax.numpy as jnp
import numpy as np
from jax import lax
from jax.experimental import pallas as pl
from jax.experimental.pallas import tpu as pltpu
from jax.experimental.pallas import tpu_sc as plsc

B, T, N = 2, 4, 4096
P = 1024
K = 32
C1, C2 = 32, 64
R2 = np.float32(0.3 * 0.3)
NCHUNK = N // 16
CAP = 96
BTN = B * T * N
BTP = B * T * P
NU = 10
PPT = 128
NUNIT = NU * B * (P // PPT)

_mesh = plsc.VectorSubcoreMesh(core_axis_name="c", subcore_axis_name="s")
_sc_params = pltpu.CompilerParams(needs_layout_passes=False)


QN = N // 4
QC = QN // 16


@functools.partial(
    pl.kernel,
    out_type=[
        jax.ShapeDtypeStruct((3 * BTN,), jnp.float32),
        jax.ShapeDtypeStruct((3 * BTP,), jnp.float32),
    ],
    mesh=_mesh,
    compiler_params=_sc_params,
    scratch_types=[
        pltpu.VMEM((3 * N,), jnp.float32),
        pltpu.VMEM((N,), jnp.float32),
        pltpu.VMEM((N,), jnp.float32),
        pltpu.VMEM((N,), jnp.float32),
        pltpu.VMEM((QN,), jnp.float32),
        pltpu.VMEM((P,), jnp.int32),
        pltpu.VMEM((P,), jnp.float32),
        pltpu.VMEM((P,), jnp.float32),
        pltpu.VMEM((P,), jnp.float32),
        pltpu.VMEM((32,), jnp.float32),
        pltpu.VMEM((512,), jnp.float32),
        pltpu.VMEM_SHARED((4096,), jnp.float32),
    ],
)
def _fps_kernel(xyzs, soa_out, axyz_out, aos_v, sx_v, sy_v, sz_v, dist_v,
                aidx_v, ax_v, ay_v, az_v, pkt_v, blk_v, shx_v):
    c_ax = lax.axis_index("c")
    s_ax = lax.axis_index("s")
    tl = s_ax // 4
    mem = s_ax % 4
    bt = c_ax * 4 + tl
    base = mem * QN
    iota = lax.iota(jnp.int32, 16)

    pltpu.sync_copy(xyzs.at[pl.ds(bt * 3 * N, 3 * N)], aos_v)

    def soa_body(c, _):
        rows = (c * 16 + iota) * 3
        for ch, ref in ((0, sx_v), (1, sy_v), (2, sz_v)):
            ref[pl.ds(c * 16, 16)] = plsc.load_gather(aos_v, [rows + ch])
        return 0

    lax.fori_loop(0, NCHUNK, soa_body, 0)

    def init_body(c, _):
        dist_v[pl.ds(c * 16, 16)] = jnp.full((16,), 1e10, jnp.float32)
        return 0

    lax.fori_loop(0, QC, init_body, 0)
    lane0 = iota == 0
    gchunk0 = mem * QC

    def step(i, far):
        farv = jnp.full((16,), far, jnp.int32)
        plsc.store_scatter(aidx_v, [jnp.full((16,), i, jnp.int32)],
                           farv, mask=lane0)
        cx = plsc.load_gather(sx_v, [farv])
        cy = plsc.load_gather(sy_v, [farv])
        cz = plsc.load_gather(sz_v, [farv])

        def chunk(c, carry):
            m, mi = carry
            s = pl.ds(base + c * 16, 16)
            dx = sx_v[s] - cx
            dy = sy_v[s] - cy
            dz = sz_v[s] - cz
            d2 = (jnp.minimum(dx * dx, 3.0) + jnp.minimum(dz * dz, 3.0)
                  ) + jnp.minimum(dy * dy, 3.0)
            dv = jnp.minimum(dist_v[pl.ds(c * 16, 16)], d2)
            dist_v[pl.ds(c * 16, 16)] = dv
            upd = dv > m
            m = jnp.where(upd, dv, m)
            mi = jnp.where(upd, jnp.full((16,), c, jnp.int32), mi)
            return m, mi

        m0 = jnp.full((16,), -1.0, jnp.float32)
        i0 = jnp.zeros((16,), jnp.int32)
        m, mi = lax.fori_loop(0, QC, chunk, (m0, i0), unroll=4)
        gidx = (mi + gchunk0) * 16 + iota
        pkt_v[pl.ds(0, 16)] = m
        pkt_v[pl.ds(16, 16)] = plsc.bitcast(gidx, jnp.float32)
        parity = (i % 2) * 2048
        pltpu.sync_copy(pkt_v,
                        shx_v.at[pl.ds(parity + tl * 512 + mem * 32, 32)])
        plsc.subcore_barrier()
        pltpu.sync_copy(shx_v.at[pl.ds(parity + tl * 512, 128)],
                        blk_v.at[pl.ds(0, 128)])
        vals = [blk_v[pl.ds(j * 32, 16)] for j in range(4)]
        idxs = [plsc.bitcast(blk_v[pl.ds(j * 32 + 16, 16)], jnp.int32)
                for j in range(4)]
        mm = jnp.maximum(jnp.maximum(vals[0], vals[1]),
                         jnp.maximum(vals[2], vals[3]))
        mx = jnp.max(mm)
        big = jnp.int32(1 << 30)
        cands = [jnp.where(vals[j] == mx, idxs[j], big) for j in range(4)]
        cand = jnp.minimum(jnp.minimum(cands[0], cands[1]),
                           jnp.minimum(cands[2], cands[3]))
        return jnp.min(cand)

    lax.fori_loop(0, P, step, jnp.int32(0))

    @pl.when(mem == 0)
    def _():
        def gath(c, _):
            s = pl.ds(c * 16, 16)
            idxs = aidx_v[s]
            ax_v[s] = plsc.load_gather(sx_v, [idxs])
            ay_v[s] = plsc.load_gather(sy_v, [idxs])
            az_v[s] = plsc.load_gather(sz_v, [idxs])
            return 0

        lax.fori_loop(0, P // 16, gath, 0)

        for ch, (src, asrc) in enumerate(((sx_v, ax_v), (sy_v, ay_v),
                                          (sz_v, az_v))):
            pltpu.sync_copy(src, soa_out.at[pl.ds(ch * BTN + bt * N, N)])
            pltpu.sync_copy(asrc, axyz_out.at[pl.ds(ch * BTP + bt * P, P)])


@functools.partial(
    pl.kernel,
    out_type=jax.ShapeDtypeStruct((NU * B * 3 * K, P), jnp.float32),
    mesh=_mesh,
    compiler_params=_sc_params,
    scratch_types=[
        pltpu.VMEM((N,), jnp.float32),
        pltpu.VMEM((N,), jnp.float32),
        pltpu.VMEM((N,), jnp.float32),
        pltpu.VMEM((PPT,), jnp.float32),
        pltpu.VMEM((PPT,), jnp.float32),
        pltpu.VMEM((PPT,), jnp.float32),
        pltpu.VMEM((CAP,), jnp.float32),
        pltpu.VMEM((CAP,), jnp.float32),
        pltpu.VMEM((CAP,), jnp.float32),
        pltpu.VMEM((3 * K, PPT), jnp.float32),
    ],
)
def _bq_kernel(soa, axyz, disp_out, nx_v, ny_v, nz_v,
               anx_v, any_v, anz_v, cpx_v, cpy_v, cpz_v, stg_v):
    wid = lax.axis_index("s") * 2 + lax.axis_index("c")
    iota = lax.iota(jnp.int32, 16)

    def job(j, _):
        unit = j * 32 + wid
        q = unit // (P // PPT)
        p0 = (unit % (P // PPT)) * PPT
        b = q % 2
        qu = q // 2
        a = (qu + 1) // 3
        u = qu - 2 * a
        pltpu.sync_copy(soa.at[pl.ds((b * T + u) * N, N)], nx_v)
        pltpu.sync_copy(soa.at[pl.ds(BTN + (b * T + u) * N, N)], ny_v)
        pltpu.sync_copy(soa.at[pl.ds(2 * BTN + (b * T + u) * N, N)], nz_v)
        abase = (b * T + a) * P + p0
        pltpu.sync_copy(axyz.at[pl.ds(abase, PPT)], anx_v)
        pltpu.sync_copy(axyz.at[pl.ds(BTP + abase, PPT)], any_v)
        pltpu.sync_copy(axyz.at[pl.ds(2 * BTP + abase, PPT)], anz_v)

        def anchor(p, _):
            pv = jnp.full((16,), p, jnp.int32)
            ax = plsc.load_gather(anx_v, [pv])
            ay = plsc.load_gather(any_v, [pv])
            az = plsc.load_gather(anz_v, [pv])

            def cond(carry):
                cnt, c = carry
                return (cnt < K) & (c < NCHUNK)

            def scan(carry):
                cnt, c = carry
                for half in range(4):
                    s = pl.ds((c + half) * 16, 16)
                    dx = nx_v[s] - ax
                    dy = ny_v[s] - ay
                    dz = nz_v[s] - az
                    d2 = (jnp.minimum(dx * dx, 3.0)
                          + jnp.minimum(dy * dy, 3.0)
                          ) + jnp.minimum(dz * dz, 3.0)
                    msk = d2 < R2
                    plsc.store_compressed(cpx_v.at[pl.ds(cnt, 16)], dx,
                                          mask=msk)
                    plsc.store_compressed(cpy_v.at[pl.ds(cnt, 16)], dy,
                                          mask=msk)
                    plsc.store_compressed(cpz_v.at[pl.ds(cnt, 16)], dz,
                                          mask=msk)
                    cnt = cnt + plsc.all_reduce_population_count(msk)[0]
                return cnt, c + 4

            cnt, _c = lax.while_loop(cond, scan, (jnp.int32(0), jnp.int32(0)))
            zero16 = jnp.zeros((16,), jnp.int32)
            hasv = jnp.broadcast_to(cnt > 0, (16,))
            cntv = jnp.broadcast_to(cnt, (16,))
            pdx = jnp.where(hasv, plsc.load_gather(cpx_v, [zero16]),
                            plsc.load_gather(nx_v, [zero16]) - ax)
            pdy = jnp.where(hasv, plsc.load_gather(cpy_v, [zero16]),
                            plsc.load_gather(ny_v, [zero16]) - ay)
            pdz = jnp.where(hasv, plsc.load_gather(cpz_v, [zero16]),
                            plsc.load_gather(nz_v, [zero16]) - az)
            for chunk in range(K // 16):
                lanes = iota + chunk * 16
                fill = lanes >= cntv
                for ch, (ref, padv) in enumerate(((cpx_v, pdx), (cpy_v, pdy),
                                                  (cpz_v, pdz))):
                    v = ref[pl.ds(chunk * 16, 16)]
                    v = jnp.where(fill, padv, v)
                    rows = lanes + ch * K
                    plsc.store_scatter(stg_v, [rows, pv], v)
            return 0

        lax.fori_loop(0, PPT, anchor, 0)
        row0 = q * 3 * K
        pltpu.sync_copy(stg_v,
                        disp_out.at[pl.ds(row0, 3 * K), pl.ds(p0, PPT)])
        return 0

    lax.fori_loop(0, NUNIT // 32, job, 0)


def _uq_lookup(t, w):
    return jnp.clip(t * 3 + w - 1, 0, NU - 1)


def _mlp_body(d_ref, wd_ref, wm_ref, o_ref):
    w = pl.program_id(2)
    x = d_ref[0, 0]
    wd = wd_ref[...]
    bias = (w.astype(jnp.float32) - 1.0) * wd[:, 3:4]
    h = (wd[:, 0:1] * x[0:1, :] + wd[:, 1:2] * x[1:2, :]
         + wd[:, 2:3] * x[2:3, :] + bias)
    h = jnp.maximum(h, 0.0)
    f = jnp.dot(wm_ref[...], h, preferred_element_type=jnp.float32)
    f = jnp.maximum(f, 0.0)
    mx = f[:, 0:P]
    for k in range(1, K):
        mx = jnp.maximum(mx, f[:, k * P:(k + 1) * P])

    @pl.when(w == 0)
    def _():
        o_ref[0, 0] = mx

    @pl.when(w > 0)
    def _():
        o_ref[0, 0] += mx


def _mlp_call(dispf, Wd, Wm):
    return pl.pallas_call(
        _mlp_body,
        grid=(B, T, 3),
        in_specs=[
            pl.BlockSpec((1, 1, 3, K * P),
                         lambda b, t, w: (_uq_lookup(t, w), b, 0, 0)),
            pl.BlockSpec((C1, 4), lambda b, t, w: (0, 0)),
            pl.BlockSpec((C2, C1), lambda b, t, w: (0, 0)),
        ],
        out_specs=pl.BlockSpec((1, 1, C2, P), lambda b, t, w: (b, t, 0, 0)),
        out_shape=jax.ShapeDtypeStruct((B, T, C2, P), jnp.float32),
    )(dispf, Wd, Wm)


def kernel(xyzs, Wd, Wm):
    soa, axyz = _fps_kernel(xyzs.reshape(-1))
    disp = _bq_kernel(soa, axyz)
    feats = _mlp_call(disp.reshape(NU, B, 3, K * P), Wd, Wm)
    new_xyzs = jnp.transpose(axyz.reshape(3, B, T, P), (1, 2, 3, 0))
    return new_xyzs, feats


# speedup vs baseline: 1.0512x; 1.0053x over previous
import functools

import jax
import jax.numpy as jnp
import numpy as np
from jax import lax
from jax.experimental import pallas as pl
from jax.experimental.pallas import tpu as pltpu
from jax.experimental.pallas import tpu_sc as plsc

B, T, N = 2, 4, 4096
P = 1024
K = 32
C1, C2 = 32, 64
R2 = np.float32(0.3 * 0.3)
NCHUNK = N // 16
CAP = 160
BTN = B * T * N
BTP = B * T * P
NU = 10
PPT = 128
NUNIT = NU * B * (P // PPT)

_mesh = plsc.VectorSubcoreMesh(core_axis_name="c", subcore_axis_name="s")
_sc_params = pltpu.CompilerParams(needs_layout_passes=False)


QN = N // 4
QC = QN // 16


@functools.partial(
    pl.kernel,
    out_type=[
        jax.ShapeDtypeStruct((3 * BTN,), jnp.float32),
        jax.ShapeDtypeStruct((3 * BTP,), jnp.float32),
    ],
    mesh=_mesh,
    compiler_params=_sc_params,
    scratch_types=[
        pltpu.VMEM((3 * N,), jnp.float32),
        pltpu.VMEM((N,), jnp.float32),
        pltpu.VMEM((N,), jnp.float32),
        pltpu.VMEM((N,), jnp.float32),
        pltpu.VMEM((QN,), jnp.float32),
        pltpu.VMEM((P,), jnp.int32),
        pltpu.VMEM((P,), jnp.float32),
        pltpu.VMEM((P,), jnp.float32),
        pltpu.VMEM((P,), jnp.float32),
        pltpu.VMEM((32,), jnp.float32),
        pltpu.VMEM((512,), jnp.float32),
        pltpu.VMEM_SHARED((4096,), jnp.float32),
    ],
)
def _fps_kernel(xyzs, soa_out, axyz_out, aos_v, sx_v, sy_v, sz_v, dist_v,
                aidx_v, ax_v, ay_v, az_v, pkt_v, blk_v, shx_v):
    c_ax = lax.axis_index("c")
    s_ax = lax.axis_index("s")
    tl = s_ax // 4
    mem = s_ax % 4
    bt = c_ax * 4 + tl
    base = mem * QN
    iota = lax.iota(jnp.int32, 16)

    pltpu.sync_copy(xyzs.at[pl.ds(bt * 3 * N, 3 * N)], aos_v)

    def soa_body(c, _):
        rows = (c * 16 + iota) * 3
        for ch, ref in ((0, sx_v), (1, sy_v), (2, sz_v)):
            ref[pl.ds(c * 16, 16)] = plsc.load_gather(aos_v, [rows + ch])
        return 0

    lax.fori_loop(0, NCHUNK, soa_body, 0)

    def init_body(c, _):
        dist_v[pl.ds(c * 16, 16)] = jnp.full((16,), 1e10, jnp.float32)
        return 0

    lax.fori_loop(0, QC, init_body, 0)
    lane0 = iota == 0
    gchunk0 = mem * QC

    def step(i, far):
        farv = jnp.full((16,), far, jnp.int32)
        plsc.store_scatter(aidx_v, [jnp.full((16,), i, jnp.int32)],
                           farv, mask=lane0)
        cx = plsc.load_gather(sx_v, [farv])
        cy = plsc.load_gather(sy_v, [farv])
        cz = plsc.load_gather(sz_v, [farv])

        def chunk(c, carry):
            m, mi = carry
            s = pl.ds(base + c * 16, 16)
            dx = sx_v[s] - cx
            dy = sy_v[s] - cy
            dz = sz_v[s] - cz
            d2 = (jnp.minimum(dx * dx, 3.0) + jnp.minimum(dz * dz, 3.0)
                  ) + jnp.minimum(dy * dy, 3.0)
            dv = jnp.minimum(dist_v[pl.ds(c * 16, 16)], d2)
            dist_v[pl.ds(c * 16, 16)] = dv
            upd = dv > m
            m = jnp.where(upd, dv, m)
            mi = jnp.where(upd, jnp.full((16,), c, jnp.int32), mi)
            return m, mi

        m0 = jnp.full((16,), -1.0, jnp.float32)
        i0 = jnp.zeros((16,), jnp.int32)
        m, mi = lax.fori_loop(0, QC, chunk, (m0, i0), unroll=4)
        gidx = (mi + gchunk0) * 16 + iota
        pkt_v[pl.ds(0, 16)] = m
        pkt_v[pl.ds(16, 16)] = plsc.bitcast(gidx, jnp.float32)
        parity = (i % 2) * 2048
        pltpu.sync_copy(pkt_v,
                        shx_v.at[pl.ds(parity + tl * 512 + mem * 32, 32)])
        plsc.subcore_barrier()
        pltpu.sync_copy(shx_v.at[pl.ds(parity + tl * 512, 128)],
                        blk_v.at[pl.ds(0, 128)])
        vals = [blk_v[pl.ds(j * 32, 16)] for j in range(4)]
        idxs = [plsc.bitcast(blk_v[pl.ds(j * 32 + 16, 16)], jnp.int32)
                for j in range(4)]
        mm = jnp.maximum(jnp.maximum(vals[0], vals[1]),
                         jnp.maximum(vals[2], vals[3]))
        mx = jnp.max(mm)
        big = jnp.int32(1 << 30)
        cands = [jnp.where(vals[j] == mx, idxs[j], big) for j in range(4)]
        cand = jnp.minimum(jnp.minimum(cands[0], cands[1]),
                           jnp.minimum(cands[2], cands[3]))
        return jnp.min(cand)

    lax.fori_loop(0, P, step, jnp.int32(0))

    @pl.when(mem == 0)
    def _():
        def gath(c, _):
            s = pl.ds(c * 16, 16)
            idxs = aidx_v[s]
            ax_v[s] = plsc.load_gather(sx_v, [idxs])
            ay_v[s] = plsc.load_gather(sy_v, [idxs])
            az_v[s] = plsc.load_gather(sz_v, [idxs])
            return 0

        lax.fori_loop(0, P // 16, gath, 0)

        for ch, (src, asrc) in enumerate(((sx_v, ax_v), (sy_v, ay_v),
                                          (sz_v, az_v))):
            pltpu.sync_copy(src, soa_out.at[pl.ds(ch * BTN + bt * N, N)])
            pltpu.sync_copy(asrc, axyz_out.at[pl.ds(ch * BTP + bt * P, P)])


@functools.partial(
    pl.kernel,
    out_type=jax.ShapeDtypeStruct((NU * B * 3 * K, P), jnp.float32),
    mesh=_mesh,
    compiler_params=_sc_params,
    scratch_types=[
        pltpu.VMEM((N,), jnp.float32),
        pltpu.VMEM((N,), jnp.float32),
        pltpu.VMEM((N,), jnp.float32),
        pltpu.VMEM((PPT,), jnp.float32),
        pltpu.VMEM((PPT,), jnp.float32),
        pltpu.VMEM((PPT,), jnp.float32),
        pltpu.VMEM((CAP,), jnp.float32),
        pltpu.VMEM((CAP,), jnp.float32),
        pltpu.VMEM((CAP,), jnp.float32),
        pltpu.VMEM((3 * K, PPT), jnp.float32),
    ],
)
def _bq_kernel(soa, axyz, disp_out, nx_v, ny_v, nz_v,
               anx_v, any_v, anz_v, cpx_v, cpy_v, cpz_v, stg_v):
    wid = lax.axis_index("s") * 2 + lax.axis_index("c")
    iota = lax.iota(jnp.int32, 16)

    def job(j, _):
        unit = j * 32 + wid
        q = unit // (P // PPT)
        p0 = (unit % (P // PPT)) * PPT
        b = q % 2
        qu = q // 2
        a = (qu + 1) // 3
        u = qu - 2 * a
        pltpu.sync_copy(soa.at[pl.ds((b * T + u) * N, N)], nx_v)
        pltpu.sync_copy(soa.at[pl.ds(BTN + (b * T + u) * N, N)], ny_v)
        pltpu.sync_copy(soa.at[pl.ds(2 * BTN + (b * T + u) * N, N)], nz_v)
        abase = (b * T + a) * P + p0
        pltpu.sync_copy(axyz.at[pl.ds(abase, PPT)], anx_v)
        pltpu.sync_copy(axyz.at[pl.ds(BTP + abase, PPT)], any_v)
        pltpu.sync_copy(axyz.at[pl.ds(2 * BTP + abase, PPT)], anz_v)

        def anchor(p, _):
            pv = jnp.full((16,), p, jnp.int32)
            ax = plsc.load_gather(anx_v, [pv])
            ay = plsc.load_gather(any_v, [pv])
            az = plsc.load_gather(anz_v, [pv])

            def cond(carry):
                cnt, c = carry
                return (cnt < K) & (c < NCHUNK)

            def scan(carry):
                cnt, c = carry
                for half in range(8):
                    s = pl.ds((c + half) * 16, 16)
                    dx = nx_v[s] - ax
                    dy = ny_v[s] - ay
                    dz = nz_v[s] - az
                    d2 = (jnp.minimum(dx * dx, 3.0)
                          + jnp.minimum(dy * dy, 3.0)
                          ) + jnp.minimum(dz * dz, 3.0)
                    msk = d2 < R2
                    plsc.store_compressed(cpx_v.at[pl.ds(cnt, 16)], dx,
                                          mask=msk)
                    plsc.store_compressed(cpy_v.at[pl.ds(cnt, 16)], dy,
                                          mask=msk)
                    plsc.store_compressed(cpz_v.at[pl.ds(cnt, 16)], dz,
                                          mask=msk)
                    cnt = cnt + plsc.all_reduce_population_count(msk)[0]
                return cnt, c + 8

            cnt, _c = lax.while_loop(cond, scan, (jnp.int32(0), jnp.int32(0)))
            zero16 = jnp.zeros((16,), jnp.int32)
            hasv = jnp.broadcast_to(cnt > 0, (16,))
            cntv = jnp.broadcast_to(cnt, (16,))
            pdx = jnp.where(hasv, plsc.load_gather(cpx_v, [zero16]),
                            plsc.load_gather(nx_v, [zero16]) - ax)
            pdy = jnp.where(hasv, plsc.load_gather(cpy_v, [zero16]),
                            plsc.load_gather(ny_v, [zero16]) - ay)
            pdz = jnp.where(hasv, plsc.load_gather(cpz_v, [zero16]),
                            plsc.load_gather(nz_v, [zero16]) - az)
            for chunk in range(K // 16):
                lanes = iota + chunk * 16
                fill = lanes >= cntv
                for ch, (ref, padv) in enumerate(((cpx_v, pdx), (cpy_v, pdy),
                                                  (cpz_v, pdz))):
                    v = ref[pl.ds(chunk * 16, 16)]
                    v = jnp.where(fill, padv, v)
                    rows = lanes + ch * K
                    plsc.store_scatter(stg_v, [rows, pv], v)
            return 0

        lax.fori_loop(0, PPT, anchor, 0)
        row0 = q * 3 * K
        pltpu.sync_copy(stg_v,
                        disp_out.at[pl.ds(row0, 3 * K), pl.ds(p0, PPT)])
        return 0

    lax.fori_loop(0, NUNIT // 32, job, 0)


def _uq_lookup(t, w):
    return jnp.clip(t * 3 + w - 1, 0, NU - 1)


def _mlp_body(d_ref, wd_ref, wm_ref, o_ref):
    w = pl.program_id(2)
    x = d_ref[0, 0]
    wd = wd_ref[...]
    bias = (w.astype(jnp.float32) - 1.0) * wd[:, 3:4]
    h = (wd[:, 0:1] * x[0:1, :] + wd[:, 1:2] * x[1:2, :]
         + wd[:, 2:3] * x[2:3, :] + bias)
    h = jnp.maximum(h, 0.0)
    f = jnp.dot(wm_ref[...], h, preferred_element_type=jnp.float32)
    f = jnp.maximum(f, 0.0)
    mx = f[:, 0:P]
    for k in range(1, K):
        mx = jnp.maximum(mx, f[:, k * P:(k + 1) * P])

    @pl.when(w == 0)
    def _():
        o_ref[0, 0] = mx

    @pl.when(w > 0)
    def _():
        o_ref[0, 0] += mx


def _mlp_call(dispf, Wd, Wm):
    return pl.pallas_call(
        _mlp_body,
        grid=(B, T, 3),
        in_specs=[
            pl.BlockSpec((1, 1, 3, K * P),
                         lambda b, t, w: (_uq_lookup(t, w), b, 0, 0)),
            pl.BlockSpec((C1, 4), lambda b, t, w: (0, 0)),
            pl.BlockSpec((C2, C1), lambda b, t, w: (0, 0)),
        ],
        out_specs=pl.BlockSpec((1, 1, C2, P), lambda b, t, w: (b, t, 0, 0)),
        out_shape=jax.ShapeDtypeStruct((B, T, C2, P), jnp.float32),
    )(dispf, Wd, Wm)


def kernel(xyzs, Wd, Wm):
    soa, axyz = _fps_kernel(xyzs.reshape(-1))
    disp = _bq_kernel(soa, axyz)
    feats = _mlp_call(disp.reshape(NU, B, 3, K * P), Wd, Wm)
    new_xyzs = jnp.transpose(axyz.reshape(3, B, T, P), (1, 2, 3, 0))
    return new_xyzs, feats
